# baseline (speedup 1.0000x reference)
import functools

import jax
import jax.numpy as jnp
from jax.experimental import pallas as pl
from jax.experimental.pallas import tpu as pltpu

_NBUF = 4


def _compute_slab(vb, w, *, S, d):
    cols = jax.lax.broadcasted_iota(jnp.int32, (d, S), 1)
    values = jnp.max(vb, axis=1, keepdims=True)
    idx = jnp.min(jnp.where(vb == values, cols, S), axis=1, keepdims=True)
    v_cls = vb[:, 0:1]
    out = jnp.where(cols == idx, v_cls, vb)
    out = out * w
    return jnp.where(cols == 0, values * w[0:1, 0:1], out)


def _swd7_body(m_ref, v_hbm, o_hbm, ibuf, obuf, isem, osem, *, N, S, d):
    i = pl.program_id(0)
    w = 1.0 - m_ref[0]

    @pl.when(i == 0)
    def _prologue():
        pltpu.make_async_copy(v_hbm.at[0], ibuf.at[0], isem.at[0]).start()
        pltpu.make_async_copy(v_hbm.at[1], ibuf.at[1], isem.at[1]).start()
        pltpu.make_async_copy(v_hbm.at[2], ibuf.at[2], isem.at[2]).start()

    @pl.when(i + 3 < N)
    def _prefetch():
        s = (i + 3) % _NBUF
        pltpu.make_async_copy(v_hbm.at[i + 3], ibuf.at[s], isem.at[s]).start()

    pltpu.make_async_copy(
        v_hbm.at[i], ibuf.at[i % _NBUF], isem.at[i % _NBUF]).wait()

    @pl.when(i >= _NBUF)
    def _drain_old_store():
        pltpu.make_async_copy(obuf.at[i % _NBUF], o_hbm.at[i - _NBUF],
                              osem.at[i % _NBUF]).wait()

    obuf[i % _NBUF] = _compute_slab(ibuf[i % _NBUF], w, S=S, d=d)
    pltpu.make_async_copy(obuf.at[i % _NBUF], o_hbm.at[i],
                          osem.at[i % _NBUF]).start(priority=1)

    @pl.when(i == N - 1)
    def _epilogue():
        for lag in range(min(_NBUF, N) - 1, -1, -1):
            j = i - lag
            pltpu.make_async_copy(obuf.at[j % _NBUF], o_hbm.at[j],
                                  osem.at[j % _NBUF]).wait()


def kernel(q, k, v, attn_mask):
    del q, k
    B, H, S, d = v.shape
    SPLIT = 2
    hd = d // SPLIT
    N = B * H * SPLIT
    vt = jnp.swapaxes(v, 2, 3).reshape(N, hd, S)
    mf = attn_mask.astype(jnp.float32).reshape(B * H, 1, S)
    out = pl.pallas_call(
        functools.partial(_swd7_body, N=N, S=S, d=hd),
        grid=(N,),
        in_specs=[
            pl.BlockSpec((1, 1, S), lambda i: (i // SPLIT, 0, 0)),
            pl.BlockSpec(memory_space=pl.ANY),
        ],
        out_specs=pl.BlockSpec(memory_space=pl.ANY),
        out_shape=jax.ShapeDtypeStruct((N, hd, S), v.dtype),
        scratch_shapes=[
            pltpu.VMEM((_NBUF, hd, S), v.dtype),
            pltpu.VMEM((_NBUF, hd, S), v.dtype),
            pltpu.SemaphoreType.DMA((_NBUF,)),
            pltpu.SemaphoreType.DMA((_NBUF,)),
        ],
    )(mf, vt)
    return jnp.swapaxes(out.reshape(B, H, d, S), 2, 3)


# speedup vs baseline: 1.2163x; 1.2163x over previous
import functools

import jax
import jax.numpy as jnp
from jax.experimental import pallas as pl
from jax.experimental.pallas import tpu as pltpu

_NBUF = 4


def _compute_slab(vb, w, *, S, d):
    cols = jax.lax.broadcasted_iota(jnp.int32, (d, S), 1)
    values = jnp.max(vb, axis=1, keepdims=True)
    idx = jnp.min(jnp.where(vb == values, cols, S), axis=1,
                  keepdims=True)
    v_cls = vb[:, 0:1]
    out = jnp.where(cols == idx, v_cls, vb)
    out = out * w
    return jnp.where(cols == 0, values * w[0:1, 0:1], out)


def _swd7_body(m_ref, v_hbm, o_hbm, ibuf, obuf, isem, osem, *, N, S, d):
    i = pl.program_id(0)
    w = 1.0 - m_ref[0]

    @pl.when(i == 0)
    def _prologue():
        pltpu.make_async_copy(v_hbm.at[0], ibuf.at[0], isem.at[0]).start()
        pltpu.make_async_copy(v_hbm.at[1], ibuf.at[1], isem.at[1]).start()
        pltpu.make_async_copy(v_hbm.at[2], ibuf.at[2], isem.at[2]).start()

    @pl.when(i + 3 < N)
    def _prefetch():
        s = (i + 3) % _NBUF
        pltpu.make_async_copy(v_hbm.at[i + 3], ibuf.at[s], isem.at[s]).start()

    pltpu.make_async_copy(
        v_hbm.at[i], ibuf.at[i % _NBUF], isem.at[i % _NBUF]).wait()

    @pl.when(i >= _NBUF)
    def _drain_old_store():
        pltpu.make_async_copy(obuf.at[i % _NBUF], o_hbm.at[i - _NBUF],
                              osem.at[i % _NBUF]).wait()

    obuf[i % _NBUF] = _compute_slab(ibuf[i % _NBUF], w, S=S, d=d)
    pltpu.make_async_copy(obuf.at[i % _NBUF], o_hbm.at[i],
                          osem.at[i % _NBUF]).start(priority=1)

    @pl.when(i == N - 1)
    def _epilogue():
        for lag in range(min(_NBUF, N) - 1, -1, -1):
            j = i - lag
            pltpu.make_async_copy(obuf.at[j % _NBUF], o_hbm.at[j],
                                  osem.at[j % _NBUF]).wait()


def kernel(q, k, v, attn_mask):
    del q, k
    B, H, S, d = v.shape
    N = B * H
    vt = jnp.swapaxes(v, 2, 3).reshape(N, d, S)
    mf = attn_mask.astype(jnp.float32).reshape(N, 1, S)
    out = pl.pallas_call(
        functools.partial(_swd7_body, N=N, S=S, d=d),
        grid=(N,),
        in_specs=[
            pl.BlockSpec((1, 1, S), lambda i: (i, 0, 0)),
            pl.BlockSpec(memory_space=pl.ANY),
        ],
        out_specs=pl.BlockSpec(memory_space=pl.ANY),
        out_shape=jax.ShapeDtypeStruct((N, d, S), v.dtype),
        scratch_shapes=[
            pltpu.VMEM((_NBUF, d, S), v.dtype),
            pltpu.VMEM((_NBUF, d, S), v.dtype),
            pltpu.SemaphoreType.DMA((_NBUF,)),
            pltpu.SemaphoreType.DMA((_NBUF,)),
        ],
    )(mf, vt)
    return jnp.swapaxes(out.reshape(B, H, d, S), 2, 3)
